# Initial kernel scaffold; baseline (speedup 1.0000x reference)
import jax
import jax.numpy as jnp
from jax.experimental import pallas as pl


def kernel(pred, target, label):
    raise NotImplementedError("write your pallas kernel here")


# speedup vs baseline: 1.4299x; 1.4299x over previous
import functools

import jax
import jax.numpy as jnp
from jax import lax
from jax.experimental import pallas as pl
from jax.experimental.pallas import tpu as pltpu
from jax.experimental.pallas import tpu_sc as plsc

BINS = 30

_NC, _NS = 2, 16
_NW = _NC * _NS
_CHUNK = 128
_CPW = 50
_NPAD = _NW * _CPW * _CHUNK


def _sc_gather(target_flat, idx):
    mesh = plsc.VectorSubcoreMesh(
        core_axis_name="c", subcore_axis_name="s",
        num_cores=_NC, num_subcores=_NS)

    @functools.partial(
        pl.kernel,
        out_type=jax.ShapeDtypeStruct((_NW, _CPW, _CHUNK), jnp.float32),
        mesh=mesh,
        scratch_types=[
            pltpu.VMEM((_CPW, _CHUNK), jnp.int32),
            pltpu.VMEM((_CPW, _CHUNK), jnp.float32),
            pltpu.SemaphoreType.DMA,
        ],
    )
    def k(tgt_hbm, idx_hbm, out_hbm, idx_v, vals_v, sem):
        wid = lax.axis_index("s") * _NC + lax.axis_index("c")
        pltpu.sync_copy(idx_hbm.at[wid], idx_v)
        for g in range(0, _CPW, 25):
            cps = [pltpu.async_copy(tgt_hbm.at[idx_v.at[g + j]],
                                    vals_v.at[g + j], sem)
                   for j in range(25)]
            for cp in cps:
                cp.wait()
        pltpu.sync_copy(vals_v, out_hbm.at[wid])

    return k(target_flat, idx)


def _tc_rows(pred, label2d, block_b):
    n, c = pred.shape
    grid = n // block_b

    def body(pred_ref, lab_ref, logp_ref, sp_ref):
        p = pred_ref[...]
        lab = lab_ref[...]
        m = jnp.max(p, axis=1, keepdims=True)
        e = jnp.exp(p - m)
        s = jnp.sum(e, axis=1, keepdims=True)
        lse = m + jnp.log(s)
        cls = lax.broadcasted_iota(jnp.int32, (block_b, c), 1)
        onehot = cls == lab
        plab = jnp.sum(jnp.where(onehot, p, 0.0), axis=1, keepdims=True)
        logp_ref[...] = plab - lse
        sp_ref[...] = jax.nn.sigmoid(plab)

    return pl.pallas_call(
        body,
        grid=(grid,),
        in_specs=[
            pl.BlockSpec((block_b, c), lambda i: (i, 0)),
            pl.BlockSpec((block_b, 1), lambda i: (i, 0)),
        ],
        out_specs=[
            pl.BlockSpec((block_b, 1), lambda i: (i, 0)),
            pl.BlockSpec((block_b, 1), lambda i: (i, 0)),
        ],
        out_shape=[
            jax.ShapeDtypeStruct((n, 1), jnp.float32),
            jax.ShapeDtypeStruct((n, 1), jnp.float32),
        ],
    )(pred, label2d)


def _tc_hist_loss(logp, sp, tv, block_b):
    n = logp.shape[0]
    grid = n // block_b
    last = grid - 1

    def body(logp_ref, sp_ref, tv_ref, out_ref, acc_ref):
        i = pl.program_id(0)

        @pl.when(i == 0)
        def _():
            acc_ref[...] = jnp.zeros((8, 128), jnp.float32)

        logp = logp_ref[...]
        g = jnp.abs(sp_ref[...] - tv_ref[...])
        bidx = jnp.clip(jnp.floor(g * BINS).astype(jnp.int32), 0, BINS - 1)
        binlane = lax.broadcasted_iota(jnp.int32, (block_b, 128), 1)
        onehot = binlane == bidx
        cnt = jnp.sum(onehot.astype(jnp.float32), axis=0, keepdims=True)
        sm = jnp.sum(jnp.where(onehot, logp, 0.0), axis=0, keepdims=True)
        row = lax.broadcasted_iota(jnp.int32, (8, 128), 0)
        upd = jnp.where(row == 0, jnp.broadcast_to(cnt, (8, 128)),
                        jnp.where(row == 1, jnp.broadcast_to(sm, (8, 128)),
                                  0.0))
        acc_ref[...] = acc_ref[...] + upd

        @pl.when(i == last)
        def _():
            acc = acc_ref[...]
            c = lax.slice(acc, (0, 0), (1, 128))
            s = lax.slice(acc, (1, 0), (2, 128))
            nne = jnp.sum((c > 0).astype(jnp.float32))
            contrib = jnp.where(c > 0, s / jnp.maximum(c, 1.0), 0.0)
            loss = -jnp.sum(contrib) / jnp.maximum(nne, 1.0)
            out_ref[...] = jnp.full((8, 128), loss, jnp.float32)

    out = pl.pallas_call(
        body,
        grid=(grid,),
        in_specs=[
            pl.BlockSpec((block_b, 1), lambda i: (i, 0)),
            pl.BlockSpec((block_b, 1), lambda i: (i, 0)),
            pl.BlockSpec((block_b, 1), lambda i: (i, 0)),
        ],
        out_specs=pl.BlockSpec((8, 128), lambda i: (0, 0)),
        out_shape=jax.ShapeDtypeStruct((8, 128), jnp.float32),
        scratch_shapes=[pltpu.VMEM((8, 128), jnp.float32)],
    )(logp, sp, tv)
    return out[0, 0]


def kernel(pred, target, label):
    n, c = pred.shape
    lab = label.astype(jnp.int32)

    i_arr = jnp.arange(_NPAD, dtype=jnp.int32)
    lab_pad = jnp.concatenate(
        [lab, jnp.zeros((_NPAD - n,), jnp.int32)])
    idx = jnp.where(i_arr < n, i_arr * c + lab_pad, 0)
    idx = idx.reshape(_NW, _CPW, _CHUNK)

    tv_pad = _sc_gather(target.reshape(-1), idx)
    tv = tv_pad.reshape(_NPAD, 1)[:n]

    logp, sp = _tc_rows(pred, lab.reshape(n, 1), block_b=2000)
    return _tc_hist_loss(logp, sp, tv, block_b=8000)


# speedup vs baseline: 3.3909x; 2.3714x over previous
import jax
import jax.numpy as jnp
from jax import lax
from jax.experimental import pallas as pl
from jax.experimental.pallas import tpu as pltpu

BINS = 30
_BLOCK = 2000


def _fused(pred, target, label2d, block_b):
    n, c = pred.shape
    grid = n // block_b
    last = grid - 1

    def body(pred_ref, tgt_ref, lab_ref, out_ref, acc_ref):
        i = pl.program_id(0)

        @pl.when(i == 0)
        def _():
            acc_ref[...] = jnp.zeros((8, 128), jnp.float32)

        p = pred_ref[...]
        t = tgt_ref[...]
        lab = lab_ref[...]
        m = jnp.max(p, axis=1, keepdims=True)
        e = jnp.exp(p - m)
        s = jnp.sum(e, axis=1, keepdims=True)
        lse = m + jnp.log(s)
        cls = lax.broadcasted_iota(jnp.int32, (block_b, c), 1)
        onehot = cls == lab
        plab = jnp.sum(jnp.where(onehot, p, 0.0), axis=1, keepdims=True)
        tlab = jnp.sum(jnp.where(onehot, t, 0.0), axis=1, keepdims=True)
        logp = plab - lse
        g = jnp.abs(jax.nn.sigmoid(plab) - tlab)
        bidx = jnp.clip(jnp.floor(g * BINS).astype(jnp.int32), 0, BINS - 1)
        binlane = lax.broadcasted_iota(jnp.int32, (block_b, 128), 1)
        oh2 = binlane == bidx
        cnt = jnp.sum(oh2.astype(jnp.float32), axis=0, keepdims=True)
        sm = jnp.sum(jnp.where(oh2, logp, 0.0), axis=0, keepdims=True)
        row = lax.broadcasted_iota(jnp.int32, (8, 128), 0)
        upd = jnp.where(row == 0, jnp.broadcast_to(cnt, (8, 128)),
                        jnp.where(row == 1, jnp.broadcast_to(sm, (8, 128)),
                                  0.0))
        acc_ref[...] = acc_ref[...] + upd

        @pl.when(i == last)
        def _():
            acc = acc_ref[...]
            cb = lax.slice(acc, (0, 0), (1, 128))
            sb = lax.slice(acc, (1, 0), (2, 128))
            nne = jnp.sum((cb > 0).astype(jnp.float32))
            contrib = jnp.where(cb > 0, sb / jnp.maximum(cb, 1.0), 0.0)
            loss = -jnp.sum(contrib) / jnp.maximum(nne, 1.0)
            out_ref[...] = jnp.full((8, 128), loss, jnp.float32)

    out = pl.pallas_call(
        body,
        grid=(grid,),
        in_specs=[
            pl.BlockSpec((block_b, c), lambda i: (i, 0)),
            pl.BlockSpec((block_b, c), lambda i: (i, 0)),
            pl.BlockSpec((block_b, 1), lambda i: (i, 0)),
        ],
        out_specs=pl.BlockSpec((8, 128), lambda i: (0, 0)),
        out_shape=jax.ShapeDtypeStruct((8, 128), jnp.float32),
        scratch_shapes=[pltpu.VMEM((8, 128), jnp.float32)],
    )(pred, target, label2d)
    return out[0, 0]


def kernel(pred, target, label):
    n, c = pred.shape
    lab = label.astype(jnp.int32).reshape(n, 1)
    return _fused(pred, target, lab, _BLOCK)


# speedup vs baseline: 3.4686x; 1.0229x over previous
import jax
import jax.numpy as jnp
from jax import lax
from jax.experimental import pallas as pl
from jax.experimental.pallas import tpu as pltpu

BINS = 30
_BLOCK = 4000


def _fused(pred, target, label2d, block_b):
    n, c = pred.shape
    grid = n // block_b
    last = grid - 1

    def body(pred_ref, tgt_ref, lab_ref, out_ref, acc_ref):
        i = pl.program_id(0)

        @pl.when(i == 0)
        def _():
            acc_ref[...] = jnp.zeros((8, 128), jnp.float32)

        p = pred_ref[...]
        t = tgt_ref[...]
        lab = lab_ref[...]
        m = jnp.max(p, axis=1, keepdims=True)
        e = jnp.exp(p - m)
        s = jnp.sum(e, axis=1, keepdims=True)
        lse = m + jnp.log(s)
        cls = lax.broadcasted_iota(jnp.int32, (block_b, c), 1)
        onehot = cls == lab
        plab = jnp.sum(jnp.where(onehot, p, 0.0), axis=1, keepdims=True)
        tlab = jnp.sum(jnp.where(onehot, t, 0.0), axis=1, keepdims=True)
        logp = plab - lse
        g = jnp.abs(jax.nn.sigmoid(plab) - tlab)
        bidx = jnp.clip(jnp.floor(g * BINS).astype(jnp.int32), 0, BINS - 1)
        binlane = lax.broadcasted_iota(jnp.int32, (block_b, 128), 1)
        oh2 = binlane == bidx
        cnt = jnp.sum(oh2.astype(jnp.float32), axis=0, keepdims=True)
        sm = jnp.sum(jnp.where(oh2, logp, 0.0), axis=0, keepdims=True)
        row = lax.broadcasted_iota(jnp.int32, (8, 128), 0)
        upd = jnp.where(row == 0, jnp.broadcast_to(cnt, (8, 128)),
                        jnp.where(row == 1, jnp.broadcast_to(sm, (8, 128)),
                                  0.0))
        acc_ref[...] = acc_ref[...] + upd

        @pl.when(i == last)
        def _():
            acc = acc_ref[...]
            cb = lax.slice(acc, (0, 0), (1, 128))
            sb = lax.slice(acc, (1, 0), (2, 128))
            nne = jnp.sum((cb > 0).astype(jnp.float32))
            contrib = jnp.where(cb > 0, sb / jnp.maximum(cb, 1.0), 0.0)
            loss = -jnp.sum(contrib) / jnp.maximum(nne, 1.0)
            out_ref[...] = jnp.full((8, 128), loss, jnp.float32)

    out = pl.pallas_call(
        body,
        grid=(grid,),
        in_specs=[
            pl.BlockSpec((block_b, c), lambda i: (i, 0)),
            pl.BlockSpec((block_b, c), lambda i: (i, 0)),
            pl.BlockSpec((block_b, 1), lambda i: (i, 0)),
        ],
        out_specs=pl.BlockSpec((8, 128), lambda i: (0, 0)),
        out_shape=jax.ShapeDtypeStruct((8, 128), jnp.float32),
        scratch_shapes=[pltpu.VMEM((8, 128), jnp.float32)],
    )(pred, target, label2d)
    return out[0, 0]


def kernel(pred, target, label):
    n, c = pred.shape
    lab = label.astype(jnp.int32).reshape(n, 1)
    return _fused(pred, target, lab, _BLOCK)


# speedup vs baseline: 3.8374x; 1.1063x over previous
import functools

import jax
import jax.numpy as jnp
from jax import lax
from jax.experimental import pallas as pl
from jax.experimental.pallas import tpu as pltpu
from jax.experimental.pallas import tpu_sc as plsc

BINS = 30
_BLOCK = 4000
_NT = 136000
_NC, _NS = 2, 16
_NW = _NC * _NS
_CH = 400
_GPC = _CH // 16


def _tc_part(pred, target, label2d, nt, block_b):
    n, c = pred.shape
    grid = nt // block_b

    def body(pred_ref, tgt_ref, lab_ref, out_ref):
        i = pl.program_id(0)

        @pl.when(i == 0)
        def _():
            out_ref[...] = jnp.zeros((8, 128), jnp.float32)

        p = pred_ref[...]
        t = tgt_ref[...]
        lab = lab_ref[...]
        m = jnp.max(p, axis=1, keepdims=True)
        e = jnp.exp(p - m)
        s = jnp.sum(e, axis=1, keepdims=True)
        lse = m + jnp.log(s)
        cls = lax.broadcasted_iota(jnp.int32, (block_b, c), 1)
        onehot = cls == lab
        plab = jnp.sum(jnp.where(onehot, p, 0.0), axis=1, keepdims=True)
        tlab = jnp.sum(jnp.where(onehot, t, 0.0), axis=1, keepdims=True)
        logp = plab - lse
        g = jnp.abs(jax.nn.sigmoid(plab) - tlab)
        bidx = jnp.clip(jnp.floor(g * BINS).astype(jnp.int32), 0, BINS - 1)
        binlane = lax.broadcasted_iota(jnp.int32, (block_b, 128), 1)
        oh2 = binlane == bidx
        cnt = jnp.sum(oh2.astype(jnp.float32), axis=0, keepdims=True)
        sm = jnp.sum(jnp.where(oh2, logp, 0.0), axis=0, keepdims=True)
        row = lax.broadcasted_iota(jnp.int32, (8, 128), 0)
        upd = jnp.where(row == 0, jnp.broadcast_to(cnt, (8, 128)),
                        jnp.where(row == 1, jnp.broadcast_to(sm, (8, 128)),
                                  0.0))
        out_ref[...] = out_ref[...] + upd

    return pl.pallas_call(
        body,
        grid=(grid,),
        in_specs=[
            pl.BlockSpec((block_b, c), lambda i: (i, 0)),
            pl.BlockSpec((block_b, c), lambda i: (i, 0)),
            pl.BlockSpec((block_b, 1), lambda i: (i, 0)),
        ],
        out_specs=pl.BlockSpec((8, 128), lambda i: (0, 0)),
        out_shape=jax.ShapeDtypeStruct((8, 128), jnp.float32),
    )(pred, target, label2d)


def _sc_part(pred, target, lab, nt):
    n, c = pred.shape
    rpw = (n - nt) // _NW
    mesh = plsc.VectorSubcoreMesh(core_axis_name="c", subcore_axis_name="s",
                                  num_cores=_NC, num_subcores=_NS)
    cp = pltpu.CompilerParams(use_tc_tiling_on_sc=True,
                              needs_layout_passes=False)

    @functools.partial(
        pl.kernel,
        out_type=jax.ShapeDtypeStruct((_NW * 8, 128), jnp.float32),
        mesh=mesh,
        compiler_params=cp,
        scratch_types=[
            pltpu.VMEM((_CH, 80), jnp.float32),
            pltpu.VMEM((_CH, 80), jnp.float32),
            pltpu.VMEM((_CH,), jnp.int32),
            pltpu.VMEM((8, 128), jnp.float32),
        ],
    )
    def k(pred_hbm, tgt_hbm, lab_hbm, out_hbm, bp, bt, bl, tbl):
        wid = lax.axis_index("s") * _NC + lax.axis_index("c")
        base = nt + wid * rpw
        l16 = lax.iota(jnp.int32, 16)
        zero16 = jnp.zeros((16,), jnp.float32)
        one16 = jnp.full((16,), 1.0, jnp.float32)
        for r in range(8):
            for q in range(8):
                tbl[r, pl.ds(q * 16, 16)] = zero16
        rowsel = lax.shift_right_logical(l16, 2)
        colbase = (l16 & 3) * 32
        phase = l16 * 5
        ln2 = jnp.float32(0.6931471805599453)

        def group(g, carry):
            ridx = g * 16 + l16
            lab_v = bl[pl.ds(g * 16, 16)]
            plab = plsc.load_gather(bp, [ridx, lab_v])
            tlab = plsc.load_gather(bt, [ridx, lab_v])
            m = jnp.full((16,), -3.0e38, jnp.float32)
            for kk in range(80):
                t0 = phase + kk
                cv = jnp.where(t0 >= 80, t0 - 80, t0)
                x = plsc.load_gather(bp, [ridx, cv])
                m = jnp.maximum(m, x)
            s = jnp.zeros((16,), jnp.float32)
            for kk in range(80):
                t0 = phase + kk
                cv = jnp.where(t0 >= 80, t0 - 80, t0)
                x = plsc.load_gather(bp, [ridx, cv])
                s = s + jnp.exp(x - m)
            bits = plsc.bitcast(s, jnp.int32)
            ev = (lax.shift_right_logical(bits, 23) & 0xFF) - 127
            mant = plsc.bitcast((bits & 0x007FFFFF) | 0x3F800000, jnp.float32)
            yv = (mant - 1.0) / (mant + 1.0)
            y2 = yv * yv
            lnm = 2.0 * yv * (1.0 + y2 * (jnp.float32(1.0 / 3.0)
                                          + y2 * jnp.float32(0.2)))
            lns = ev.astype(jnp.float32) * ln2 + lnm
            logp = plab - m - lns
            sp = 1.0 / (1.0 + jnp.exp(-plab))
            gg = jnp.abs(sp - tlab)
            bi = jnp.clip((gg * BINS).astype(jnp.int32), 0, BINS - 1)
            colc = colbase + bi
            plsc.addupdate_scatter(tbl, [rowsel, colc], one16)
            plsc.addupdate_scatter(tbl, [rowsel + 4, colc], logp)
            return carry

        for chn in range(rpw // _CH):
            cb0 = base + chn * _CH
            pltpu.sync_copy(pred_hbm.at[pl.ds(cb0, _CH)], bp)
            pltpu.sync_copy(tgt_hbm.at[pl.ds(cb0, _CH)], bt)
            pltpu.sync_copy(lab_hbm.at[pl.ds(cb0, _CH)], bl)
            lax.fori_loop(0, _GPC, group, 0)
        pltpu.sync_copy(tbl, out_hbm.at[pl.ds(wid * 8, 8)])

    return k(pred, target, lab)


def _merge(acc, sc_tbl):
    def body(acc_ref, sc_ref, out_ref):
        v = sc_ref[...]
        y = jnp.zeros((8, 128), jnp.float32)
        for w in range(_NW):
            y = y + lax.slice(v, (8 * w, 0), (8 * w + 8, 128))
        cnt_row = jnp.sum(lax.slice(y, (0, 0), (4, 128)), axis=0,
                          keepdims=True)
        sm_row = jnp.sum(lax.slice(y, (4, 0), (8, 128)), axis=0,
                         keepdims=True)
        cnt32 = jnp.zeros((1, 32), jnp.float32)
        sm32 = jnp.zeros((1, 32), jnp.float32)
        for q in range(4):
            cnt32 = cnt32 + lax.slice(cnt_row, (0, 32 * q), (1, 32 * q + 32))
            sm32 = sm32 + lax.slice(sm_row, (0, 32 * q), (1, 32 * q + 32))
        acc = acc_ref[...]
        cb = cnt32 + lax.slice(acc, (0, 0), (1, 32))
        sb = sm32 + lax.slice(acc, (1, 0), (2, 32))
        nne = jnp.sum((cb > 0).astype(jnp.float32))
        contrib = jnp.where(cb > 0, sb / jnp.maximum(cb, 1.0), 0.0)
        loss = -jnp.sum(contrib) / jnp.maximum(nne, 1.0)
        out_ref[...] = jnp.full((8, 128), loss, jnp.float32)

    out = pl.pallas_call(
        body,
        out_shape=jax.ShapeDtypeStruct((8, 128), jnp.float32),
    )(acc, sc_tbl)
    return out[0, 0]


def kernel(pred, target, label):
    n, c = pred.shape
    lab = label.astype(jnp.int32)
    acc = _tc_part(pred, target, lab.reshape(n, 1), _NT, _BLOCK)
    sc_tbl = _sc_part(pred, target, lab, _NT)
    return _merge(acc, sc_tbl)
